# Initial kernel scaffold; baseline (speedup 1.0000x reference)
import jax
import jax.numpy as jnp
from jax.experimental import pallas as pl


def kernel(input_ids, token_table, position_table):
    raise NotImplementedError("write your pallas kernel here")


# speedup vs baseline: 2.7179x; 2.7179x over previous
import functools

import jax
import jax.numpy as jnp
from jax import lax
from jax.experimental import pallas as pl
from jax.experimental.pallas import tpu as pltpu
from jax.experimental.pallas import tpu_sc as plsc

B = 1024
S = 200
D = 64
VOCAB = 100000
NC = 2
NS = 16
NW = NC * NS
ROWS_PER_W = (B * S) // NW
BLK = 200
NBLK = ROWS_PER_W // BLK
LANES = 16
K = D // LANES


def _mesh():
    return plsc.VectorSubcoreMesh(
        core_axis_name="c", subcore_axis_name="s", num_cores=NC, num_subcores=NS
    )


@functools.partial(
    pl.kernel,
    out_type=jax.ShapeDtypeStruct((B * S, D), jnp.float32),
    mesh=_mesh(),
    scratch_types=[
        pltpu.VMEM((NBLK, BLK), jnp.int32),
        pltpu.VMEM((S, D), jnp.float32),
        pltpu.VMEM((BLK, D), jnp.float32),
        pltpu.SemaphoreType.DMA,
    ],
    compiler_params=pltpu.CompilerParams(use_tc_tiling_on_sc=False),
)
def _embed(ids_hbm, tok_hbm, pos_hbm, out_hbm, idx_v, pos_v, rows_v, sem):
    cid = lax.axis_index("c")
    sid = lax.axis_index("s")
    wid = sid * NC + cid
    base = wid * ROWS_PER_W

    pltpu.sync_copy(ids_hbm.at[wid], idx_v)
    pltpu.sync_copy(pos_hbm.at[pl.ds(0, S)], pos_v)

    def block(j, carry):
        pltpu.async_copy(tok_hbm.at[idx_v.at[j]], rows_v, sem).wait()

        def radd(r, c):
            for k in range(K):
                v = pos_v[r, pl.ds(k * LANES, LANES)]
                plsc.addupdate(rows_v.at[r, pl.ds(k * LANES, LANES)], v)
            return c

        lax.fori_loop(0, BLK, radd, 0)
        pltpu.sync_copy(rows_v, out_hbm.at[pl.ds(base + j * BLK, BLK)])
        return carry

    lax.fori_loop(0, NBLK, block, 0)


def kernel(input_ids, token_table, position_table):
    ids = input_ids.astype(jnp.int32).reshape(NW, NBLK, BLK)
    out = _embed(ids, token_table, position_table)
    return out.reshape(B, S, D)


# speedup vs baseline: 3.2570x; 1.1984x over previous
import functools

import jax
import jax.numpy as jnp
from jax import lax
from jax.experimental import pallas as pl
from jax.experimental.pallas import tpu as pltpu
from jax.experimental.pallas import tpu_sc as plsc

B = 1024
S = 200
D = 64
VOCAB = 100000
NC = 2
NS = 16
NW = NC * NS
ROWS_PER_W = (B * S) // NW
BLK = 200
NBLK = ROWS_PER_W // BLK
LANES = 16
K = D // LANES
NBUF = 4
AHEAD = 2
NT = NBLK // NBUF


def _mesh():
    return plsc.VectorSubcoreMesh(
        core_axis_name="c", subcore_axis_name="s", num_cores=NC, num_subcores=NS
    )


@functools.partial(
    pl.kernel,
    out_type=jax.ShapeDtypeStruct((B * S, D), jnp.float32),
    mesh=_mesh(),
    scratch_types=[
        pltpu.VMEM((NBLK, BLK), jnp.int32),
        pltpu.VMEM((S, D), jnp.float32),
        pltpu.VMEM((NBUF, BLK, D), jnp.float32),
        pltpu.SemaphoreType.DMA((NBUF,)),
        pltpu.SemaphoreType.DMA((NBUF,)),
    ],
    compiler_params=pltpu.CompilerParams(use_tc_tiling_on_sc=False),
)
def _embed(ids_hbm, tok_hbm, pos_hbm, out_hbm, idx_v, pos_v, rows_v, sem_g, sem_s):
    cid = lax.axis_index("c")
    sid = lax.axis_index("s")
    wid = sid * NC + cid
    base = wid * ROWS_PER_W

    pltpu.sync_copy(ids_hbm.at[wid], idx_v)
    pltpu.sync_copy(pos_hbm.at[pl.ds(0, S)], pos_v)

    def gather(j, b):
        return pltpu.make_async_copy(
            tok_hbm.at[idx_v.at[j]], rows_v.at[b], sem_g.at[b]
        )

    def store(j, b):
        return pltpu.make_async_copy(
            rows_v.at[b], out_hbm.at[pl.ds(base + j * BLK, BLK)], sem_s.at[b]
        )

    for j0 in range(AHEAD):
        gather(j0, j0).start()

    def t_body(t, carry):
        for b in range(NBUF):
            j = NBUF * t + b
            bn = (b + AHEAD) % NBUF

            if b < AHEAD:
                @pl.when(t >= 1)
                def _():
                    store(j - AHEAD, bn).wait()

                gather(j + AHEAD, bn).start()
            else:
                @pl.when(t < NT - 1)
                def _():
                    store(j - AHEAD, bn).wait()
                    gather(j + AHEAD, bn).start()

            gather(j, b).wait()

            def radd(r, c):
                for u in range(4):
                    rr = 4 * r + u
                    for k in range(K):
                        v = pos_v[rr, pl.ds(k * LANES, LANES)]
                        plsc.addupdate(rows_v.at[b, rr, pl.ds(k * LANES, LANES)], v)
                return c

            lax.fori_loop(0, BLK // 4, radd, 0)
            store(j, b).start()
        return carry

    lax.fori_loop(0, NT, t_body, 0)

    for b in range(NBUF):
        store(NBLK - NBUF + b, b).wait()


def kernel(input_ids, token_table, position_table):
    ids = input_ids.astype(jnp.int32).reshape(NW, NBLK, BLK)
    out = _embed(ids, token_table, position_table)
    return out.reshape(B, S, D)
